# Initial kernel scaffold; baseline (speedup 1.0000x reference)
import jax
import jax.numpy as jnp
from jax.experimental import pallas as pl


def kernel(x, w_gate, W1, b1, W2, b2):
    raise NotImplementedError("write your pallas kernel here")


# speedup vs baseline: 1.7229x; 1.7229x over previous
import functools

import jax
import jax.numpy as jnp
import numpy as np
from jax import lax
from jax.experimental import pallas as pl
from jax.experimental.pallas import tpu as pltpu
from jax.experimental.pallas import tpu_sc as plsc

_EPS = float(np.finfo(np.float64).eps)


def _gating_body(x_ref, wg_ref, i1_ref, i2_ref, g1_ref, g2_ref):
    logits = lax.dot_general(x_ref[...], wg_ref[...], (((1,), (0,)), ((), ())),
                             preferred_element_type=jnp.float32)
    e = logits.shape[1]
    col = lax.broadcasted_iota(jnp.int32, logits.shape, 1)
    m1 = jnp.max(logits, axis=1, keepdims=True)
    i1 = jnp.min(jnp.where(logits == m1, col, e), axis=1, keepdims=True)
    masked = jnp.where(col == i1, -jnp.inf, logits)
    m2 = jnp.max(masked, axis=1, keepdims=True)
    i2 = jnp.min(jnp.where(masked == m2, col, e), axis=1, keepdims=True)
    t = jnp.exp(m2 - m1)
    denom = 1.0 + t
    i1_ref[...] = i1
    i2_ref[...] = i2
    g1_ref[...] = 1.0 / denom
    g2_ref[...] = t / denom


def _gmm_body(te_ref, na_ref, xs_ref, g_ref, w1_ref, b1_ref, w2_ref, b2_ref,
              ys_ref):
    i = pl.program_id(0)

    @pl.when(i < na_ref[0])
    def _():
        h = lax.dot_general(xs_ref[...], w1_ref[0], (((1,), (0,)), ((), ())),
                            preferred_element_type=jnp.float32)
        h = jnp.maximum(h + b1_ref[0], 0.0)
        o = lax.dot_general(h, w2_ref[0], (((1,), (0,)), ((), ())),
                            preferred_element_type=jnp.float32)
        o = jnp.tanh(o + b2_ref[0]) * 10.0
        ys_ref[...] = jnp.exp(o) * g_ref[...]


def _combine_body(yt_ref, o_ref):
    s = yt_ref[:, 0, :] + yt_ref[:, 1, :]
    o_ref[...] = jnp.log(jnp.where(s == 0.0, _EPS, s))


def _sc_gather(table, idx):
    _, d = table.shape
    b = idx.shape[0]
    nw = 32
    ch = 128
    b_per_w = b // nw
    nch = b_per_w // ch
    mesh = plsc.VectorSubcoreMesh(core_axis_name="c", subcore_axis_name="s")

    @functools.partial(
        pl.kernel,
        mesh=mesh,
        out_type=jax.ShapeDtypeStruct((b, d), jnp.float32),
        scratch_types=[
            pltpu.VMEM((ch,), jnp.int32),
            pltpu.VMEM((ch, d), jnp.float32),
            pltpu.SemaphoreType.DMA,
        ],
    )
    def gather_k(table_hbm, idx_hbm, out_hbm, idxc, rows, sem):
        wid = lax.axis_index("s") * 2 + lax.axis_index("c")
        base = wid * b_per_w

        def body(c, carry):
            off = base + c * ch
            pltpu.sync_copy(idx_hbm.at[pl.ds(off, ch)], idxc)
            pltpu.async_copy(table_hbm.at[idxc], rows, sem).wait()
            pltpu.sync_copy(rows, out_hbm.at[pl.ds(off, ch)])
            return carry

        lax.fori_loop(0, nch, body, 0)

    return gather_k(table, idx)


def kernel(x, w_gate, W1, b1, W2, b2):
    n, d = x.shape
    e = w_gate.shape[1]
    h = W1.shape[2]
    out_d = W2.shape[2]
    tm = 128
    p = 2 * n + (tm - 1) * min(e, 2 * n)
    p = ((p + 255) // 256) * 256
    tn = p // tm

    tg = 512
    i1, i2, g1, g2 = pl.pallas_call(
        _gating_body,
        grid=(n // tg,),
        in_specs=[
            pl.BlockSpec((tg, d), lambda i: (i, 0)),
            pl.BlockSpec((d, e), lambda i: (0, 0)),
        ],
        out_specs=[
            pl.BlockSpec((tg, 1), lambda i: (i, 0)),
            pl.BlockSpec((tg, 1), lambda i: (i, 0)),
            pl.BlockSpec((tg, 1), lambda i: (i, 0)),
            pl.BlockSpec((tg, 1), lambda i: (i, 0)),
        ],
        out_shape=[
            jax.ShapeDtypeStruct((n, 1), jnp.int32),
            jax.ShapeDtypeStruct((n, 1), jnp.int32),
            jax.ShapeDtypeStruct((n, 1), jnp.float32),
            jax.ShapeDtypeStruct((n, 1), jnp.float32),
        ],
    )(x, w_gate)

    flat_e = jnp.concatenate([i1, i2], axis=1).reshape(-1)
    flat_g = jnp.concatenate([g1, g2], axis=1).reshape(-1)
    order = jnp.argsort(flat_e, stable=True).astype(jnp.int32)
    sorted_e = jnp.take(flat_e, order)
    counts = jnp.zeros((e,), jnp.int32).at[flat_e].add(1)
    offs = jnp.concatenate([jnp.zeros((1,), jnp.int32), jnp.cumsum(counts)])
    pcounts = ((counts + tm - 1) // tm) * tm
    poffs = jnp.concatenate([jnp.zeros((1,), jnp.int32), jnp.cumsum(pcounts)])
    j = jnp.arange(2 * n, dtype=jnp.int32)
    slot = jnp.take(poffs, sorted_e) + (j - jnp.take(offs, sorted_e))
    idx_pad = jnp.zeros((p,), jnp.int32).at[slot].set(order // 2)
    gate_pad = jnp.zeros((p,), jnp.float32).at[slot].set(jnp.take(flat_g, order))
    pos = jnp.zeros((2 * n,), jnp.int32).at[order].set(slot)
    tile_start = jnp.arange(tn, dtype=jnp.int32) * tm
    tile_expert = jnp.clip(
        jnp.searchsorted(poffs, tile_start, side="right") - 1, 0, e - 1
    ).astype(jnp.int32)
    n_active = (poffs[e] // tm).reshape(1).astype(jnp.int32)

    xs = _sc_gather(x, idx_pad)

    grid_spec = pltpu.PrefetchScalarGridSpec(
        num_scalar_prefetch=2,
        grid=(tn,),
        in_specs=[
            pl.BlockSpec((tm, d), lambda i, te, na: (i, 0)),
            pl.BlockSpec((tm, 1), lambda i, te, na: (i, 0)),
            pl.BlockSpec((1, d, h), lambda i, te, na: (te[i], 0, 0)),
            pl.BlockSpec((1, 1, h), lambda i, te, na: (te[i], 0, 0)),
            pl.BlockSpec((1, h, out_d), lambda i, te, na: (te[i], 0, 0)),
            pl.BlockSpec((1, 1, out_d), lambda i, te, na: (te[i], 0, 0)),
        ],
        out_specs=pl.BlockSpec((tm, out_d), lambda i, te, na: (i, 0)),
    )
    ys = pl.pallas_call(
        _gmm_body,
        grid_spec=grid_spec,
        out_shape=jax.ShapeDtypeStruct((p, out_d), jnp.float32),
    )(tile_expert, n_active, xs, gate_pad.reshape(p, 1), W1,
      b1.reshape(e, 1, h), W2, b2.reshape(e, 1, out_d))

    yt = _sc_gather(ys, pos).reshape(n, 2, out_d)

    tb = 512
    out = pl.pallas_call(
        _combine_body,
        grid=(n // tb,),
        in_specs=[pl.BlockSpec((tb, 2, out_d), lambda i: (i, 0, 0))],
        out_specs=pl.BlockSpec((tb, out_d), lambda i: (i, 0)),
        out_shape=jax.ShapeDtypeStruct((n, out_d), jnp.float32),
    )(yt)
    return out


# speedup vs baseline: 2.6636x; 1.5460x over previous
import functools

import jax
import jax.numpy as jnp
import numpy as np
from jax import lax
from jax.experimental import pallas as pl
from jax.experimental.pallas import tpu as pltpu
from jax.experimental.pallas import tpu_sc as plsc

_EPS = float(np.finfo(np.float64).eps)


def _gating_body(x_ref, wg_ref, i1_ref, i2_ref, g1_ref, g2_ref):
    logits = lax.dot_general(x_ref[...], wg_ref[...], (((1,), (0,)), ((), ())),
                             preferred_element_type=jnp.float32)
    e = logits.shape[1]
    col = lax.broadcasted_iota(jnp.int32, logits.shape, 1)
    m1 = jnp.max(logits, axis=1, keepdims=True)
    i1 = jnp.min(jnp.where(logits == m1, col, e), axis=1, keepdims=True)
    masked = jnp.where(col == i1, -jnp.inf, logits)
    m2 = jnp.max(masked, axis=1, keepdims=True)
    i2 = jnp.min(jnp.where(masked == m2, col, e), axis=1, keepdims=True)
    t = jnp.exp(m2 - m1)
    denom = 1.0 + t
    i1_ref[...] = i1
    i2_ref[...] = i2
    g1_ref[...] = 1.0 / denom
    g2_ref[...] = t / denom


def _gmm_body(te_ref, na_ref, xs_ref, g_ref, w1_ref, b1_ref, w2_ref, b2_ref,
              ys_ref):
    i = pl.program_id(0)

    @pl.when(i < na_ref[0])
    def _():
        h = lax.dot_general(xs_ref[...], w1_ref[0], (((1,), (0,)), ((), ())),
                            preferred_element_type=jnp.float32)
        h = jnp.maximum(h + b1_ref[0], 0.0)
        o = lax.dot_general(h, w2_ref[0], (((1,), (0,)), ((), ())),
                            preferred_element_type=jnp.float32)
        o = jnp.tanh(o + b2_ref[0]) * 10.0
        ys_ref[...] = jnp.exp(o) * g_ref[...]


def _combine_body(yt_ref, o_ref):
    s = yt_ref[:, 0, :] + yt_ref[:, 1, :]
    o_ref[...] = jnp.log(jnp.where(s == 0.0, _EPS, s))


def _sc_gather(table, idx):
    _, d = table.shape
    b = idx.shape[0]
    nw = 32
    ch = 128
    b_per_w = b // nw
    nch = b_per_w // ch
    mesh = plsc.VectorSubcoreMesh(core_axis_name="c", subcore_axis_name="s")

    @functools.partial(
        pl.kernel,
        mesh=mesh,
        out_type=jax.ShapeDtypeStruct((b, d), jnp.float32),
        scratch_types=[
            pltpu.VMEM((ch,), jnp.int32),
            pltpu.VMEM((ch, d), jnp.float32),
            pltpu.SemaphoreType.DMA,
        ],
    )
    def gather_k(table_hbm, idx_hbm, out_hbm, idxc, rows, sem):
        wid = lax.axis_index("s") * 2 + lax.axis_index("c")
        base = wid * b_per_w

        def body(c, carry):
            off = base + c * ch
            pltpu.sync_copy(idx_hbm.at[pl.ds(off, ch)], idxc)
            pltpu.async_copy(table_hbm.at[idxc], rows, sem).wait()
            pltpu.sync_copy(rows, out_hbm.at[pl.ds(off, ch)])
            return carry

        lax.fori_loop(0, nch, body, 0)

    return gather_k(table, idx)


def kernel(x, w_gate, W1, b1, W2, b2):
    n, d = x.shape
    e = w_gate.shape[1]
    h = W1.shape[2]
    out_d = W2.shape[2]
    tm = 128
    p = 2 * n + (tm - 1) * min(e, 2 * n)
    p = ((p + 255) // 256) * 256
    tn = p // tm

    tg = 512
    i1, i2, g1, g2 = pl.pallas_call(
        _gating_body,
        grid=(n // tg,),
        in_specs=[
            pl.BlockSpec((tg, d), lambda i: (i, 0)),
            pl.BlockSpec((d, e), lambda i: (0, 0)),
        ],
        out_specs=[
            pl.BlockSpec((tg, 1), lambda i: (i, 0)),
            pl.BlockSpec((tg, 1), lambda i: (i, 0)),
            pl.BlockSpec((tg, 1), lambda i: (i, 0)),
            pl.BlockSpec((tg, 1), lambda i: (i, 0)),
        ],
        out_shape=[
            jax.ShapeDtypeStruct((n, 1), jnp.int32),
            jax.ShapeDtypeStruct((n, 1), jnp.int32),
            jax.ShapeDtypeStruct((n, 1), jnp.float32),
            jax.ShapeDtypeStruct((n, 1), jnp.float32),
        ],
    )(x, w_gate)

    flat_e = jnp.concatenate([i1, i2], axis=1).reshape(-1)
    flat_g = jnp.concatenate([g1, g2], axis=1).reshape(-1)
    order = jnp.argsort(flat_e, stable=True).astype(jnp.int32)
    sorted_e = jnp.take(flat_e, order)
    counts = jnp.zeros((e,), jnp.int32).at[flat_e].add(1)
    offs = jnp.concatenate([jnp.zeros((1,), jnp.int32), jnp.cumsum(counts)])
    pcounts = ((counts + tm - 1) // tm) * tm
    poffs = jnp.concatenate([jnp.zeros((1,), jnp.int32), jnp.cumsum(pcounts)])
    j = jnp.arange(2 * n, dtype=jnp.int32)
    slot = jnp.take(poffs, sorted_e) + (j - jnp.take(offs, sorted_e))
    idx_pad = (jnp.arange(p, dtype=jnp.int32) % n).at[slot].set(order // 2)
    gate_pad = jnp.zeros((p,), jnp.float32).at[slot].set(jnp.take(flat_g, order))
    pos = jnp.zeros((2 * n,), jnp.int32).at[order].set(slot)
    tile_start = jnp.arange(tn, dtype=jnp.int32) * tm
    tile_expert = jnp.clip(
        jnp.searchsorted(poffs, tile_start, side="right") - 1, 0, e - 1
    ).astype(jnp.int32)
    n_active = (poffs[e] // tm).reshape(1).astype(jnp.int32)

    xs = _sc_gather(x, idx_pad)

    grid_spec = pltpu.PrefetchScalarGridSpec(
        num_scalar_prefetch=2,
        grid=(tn,),
        in_specs=[
            pl.BlockSpec((tm, d), lambda i, te, na: (i, 0)),
            pl.BlockSpec((tm, 1), lambda i, te, na: (i, 0)),
            pl.BlockSpec((1, d, h), lambda i, te, na: (te[i], 0, 0)),
            pl.BlockSpec((1, 1, h), lambda i, te, na: (te[i], 0, 0)),
            pl.BlockSpec((1, h, out_d), lambda i, te, na: (te[i], 0, 0)),
            pl.BlockSpec((1, 1, out_d), lambda i, te, na: (te[i], 0, 0)),
        ],
        out_specs=pl.BlockSpec((tm, out_d), lambda i, te, na: (i, 0)),
    )
    ys = pl.pallas_call(
        _gmm_body,
        grid_spec=grid_spec,
        out_shape=jax.ShapeDtypeStruct((p, out_d), jnp.float32),
    )(tile_expert, n_active, xs, gate_pad.reshape(p, 1), W1,
      b1.reshape(e, 1, h), W2, b2.reshape(e, 1, out_d))

    yt = _sc_gather(ys, pos).reshape(n, 2, out_d)

    tb = 512
    out = pl.pallas_call(
        _combine_body,
        grid=(n // tb,),
        in_specs=[pl.BlockSpec((tb, 2, out_d), lambda i: (i, 0, 0))],
        out_specs=pl.BlockSpec((tb, out_d), lambda i: (i, 0)),
        out_shape=jax.ShapeDtypeStruct((n, out_d), jnp.float32),
    )(yt)
    return out


# speedup vs baseline: 5.5710x; 2.0915x over previous
import functools

import jax
import jax.numpy as jnp
import numpy as np
from jax import lax
from jax.experimental import pallas as pl
from jax.experimental.pallas import tpu as pltpu
from jax.experimental.pallas import tpu_sc as plsc

_EPS = float(np.finfo(np.float64).eps)


def _gating_body(x_ref, wg_ref, i1_ref, i2_ref, g1_ref, g2_ref, cnt_ref,
                 acc_ref):
    i = pl.program_id(0)
    logits = lax.dot_general(x_ref[...], wg_ref[...], (((1,), (0,)), ((), ())),
                             preferred_element_type=jnp.float32)
    e = logits.shape[1]
    col = lax.broadcasted_iota(jnp.int32, logits.shape, 1)
    m1 = jnp.max(logits, axis=1, keepdims=True)
    i1 = jnp.min(jnp.where(logits == m1, col, e), axis=1, keepdims=True)
    masked = jnp.where(col == i1, -jnp.inf, logits)
    m2 = jnp.max(masked, axis=1, keepdims=True)
    i2 = jnp.min(jnp.where(masked == m2, col, e), axis=1, keepdims=True)
    t = jnp.exp(m2 - m1)
    denom = 1.0 + t
    i1_ref[...] = i1
    i2_ref[...] = i2
    g1_ref[...] = 1.0 / denom
    g2_ref[...] = t / denom
    hist = (jnp.sum((i1 == col[: i1.shape[0]]).astype(jnp.int32), axis=0,
                    keepdims=True)
            + jnp.sum((i2 == col[: i2.shape[0]]).astype(jnp.int32), axis=0,
                      keepdims=True))

    @pl.when(i == 0)
    def _():
        acc_ref[...] = jnp.zeros_like(acc_ref)

    acc_ref[0:1, :] += hist

    @pl.when(i == pl.num_programs(0) - 1)
    def _():
        cnt_ref[...] = acc_ref[0:1, :]


def _routing_body(i1_ref, i2_ref, poffs_ref, p0_ref, p1_ref, acc_ref):
    i = pl.program_id(0)
    tr = i1_ref.shape[0]
    ee = jnp.concatenate([i1_ref[...], i2_ref[...]], axis=0)
    e = poffs_ref.shape[1] - 1
    col = lax.broadcasted_iota(jnp.int32, (2 * tr, e), 1)
    onehot_b = ee == col
    onehot_f = onehot_b.astype(jnp.float32)

    @pl.when(i == 0)
    def _():
        acc_ref[...] = jnp.zeros_like(acc_ref)

    r = lax.broadcasted_iota(jnp.int32, (2 * tr, 2 * tr), 0)
    c = lax.broadcasted_iota(jnp.int32, (2 * tr, 2 * tr), 1)
    lt = (c < r).astype(jnp.float32)
    excl = lax.dot_general(lt, onehot_f, (((1,), (0,)), ((), ())),
                           preferred_element_type=jnp.float32)
    rank = jnp.sum(excl * onehot_f, axis=1, keepdims=True).astype(jnp.int32)
    base = jnp.sum(jnp.where(onehot_b, poffs_ref[0:1, :e], 0), axis=1,
                   keepdims=True)
    run = jnp.sum(jnp.where(onehot_b, acc_ref[0:1, :], 0), axis=1,
                  keepdims=True)
    slot = base + run + rank
    p0_ref[...] = slot[:tr]
    p1_ref[...] = slot[tr:]
    acc_ref[0:1, :] += jnp.sum(onehot_b.astype(jnp.int32), axis=0,
                               keepdims=True)


def _gmm_body(te_ref, na_ref, xs_ref, w1_ref, b1_ref, w2_ref, b2_ref, ys_ref):
    i = pl.program_id(0)

    @pl.when(i < na_ref[0])
    def _():
        h = lax.dot_general(xs_ref[...], w1_ref[0], (((1,), (0,)), ((), ())),
                            preferred_element_type=jnp.float32)
        h = jnp.maximum(h + b1_ref[0], 0.0)
        o = lax.dot_general(h, w2_ref[0], (((1,), (0,)), ((), ())),
                            preferred_element_type=jnp.float32)
        ys_ref[...] = jnp.exp(jnp.tanh(o + b2_ref[0]) * 10.0)


def _combine_body(y1_ref, y2_ref, g1_ref, g2_ref, o_ref):
    s = g1_ref[...] * y1_ref[...] + g2_ref[...] * y2_ref[...]
    o_ref[...] = jnp.log(jnp.where(s == 0.0, _EPS, s))


def _sc_scatter_rows(x, pos0, pos1, p):
    n, d = x.shape
    nw = 32
    ch = 128
    t_per_w = n // nw
    nch = t_per_w // ch
    mesh = plsc.VectorSubcoreMesh(core_axis_name="c", subcore_axis_name="s")

    @functools.partial(
        pl.kernel,
        mesh=mesh,
        out_type=jax.ShapeDtypeStruct((p, d), jnp.float32),
        scratch_types=[
            pltpu.VMEM((2, ch), jnp.int32),
            pltpu.VMEM((ch, d), jnp.float32),
            pltpu.SemaphoreType.DMA,
        ],
    )
    def scatter_k(x_hbm, p0_hbm, p1_hbm, xs_hbm, posb, rows, sem):
        wid = lax.axis_index("s") * 2 + lax.axis_index("c")
        base = wid * t_per_w

        def body(cc, carry):
            off = base + cc * ch
            pltpu.sync_copy(p0_hbm.at[pl.ds(off, ch)], posb.at[0])
            pltpu.sync_copy(p1_hbm.at[pl.ds(off, ch)], posb.at[1])
            pltpu.sync_copy(x_hbm.at[pl.ds(off, ch)], rows)
            cp0 = pltpu.async_copy(rows, xs_hbm.at[posb.at[0]], sem)
            cp1 = pltpu.async_copy(rows, xs_hbm.at[posb.at[1]], sem)
            cp0.wait()
            cp1.wait()
            return carry

        lax.fori_loop(0, nch, body, 0)

    return scatter_k(x, pos0, pos1)


def _sc_gather(table, idx):
    _, d = table.shape
    b = idx.shape[0]
    nw = 32
    ch = 128
    b_per_w = b // nw
    nch = b_per_w // ch
    mesh = plsc.VectorSubcoreMesh(core_axis_name="c", subcore_axis_name="s")

    @functools.partial(
        pl.kernel,
        mesh=mesh,
        out_type=jax.ShapeDtypeStruct((b, d), jnp.float32),
        scratch_types=[
            pltpu.VMEM((ch,), jnp.int32),
            pltpu.VMEM((ch, d), jnp.float32),
            pltpu.SemaphoreType.DMA,
        ],
    )
    def gather_k(table_hbm, idx_hbm, out_hbm, idxc, rows, sem):
        wid = lax.axis_index("s") * 2 + lax.axis_index("c")
        base = wid * b_per_w

        def body(c, carry):
            off = base + c * ch
            pltpu.sync_copy(idx_hbm.at[pl.ds(off, ch)], idxc)
            pltpu.async_copy(table_hbm.at[idxc], rows, sem).wait()
            pltpu.sync_copy(rows, out_hbm.at[pl.ds(off, ch)])
            return carry

        lax.fori_loop(0, nch, body, 0)

    return gather_k(table, idx)


def kernel(x, w_gate, W1, b1, W2, b2):
    n, d = x.shape
    e = w_gate.shape[1]
    h = W1.shape[2]
    out_d = W2.shape[2]
    tm = 128
    p = 2 * n + (tm - 1) * min(e, 2 * n)
    p = ((p + 255) // 256) * 256
    tn = p // tm

    tg = 512
    i1, i2, g1, g2, counts = pl.pallas_call(
        _gating_body,
        grid=(n // tg,),
        in_specs=[
            pl.BlockSpec((tg, d), lambda i: (i, 0)),
            pl.BlockSpec((d, e), lambda i: (0, 0)),
        ],
        out_specs=[
            pl.BlockSpec((tg, 1), lambda i: (i, 0)),
            pl.BlockSpec((tg, 1), lambda i: (i, 0)),
            pl.BlockSpec((tg, 1), lambda i: (i, 0)),
            pl.BlockSpec((tg, 1), lambda i: (i, 0)),
            pl.BlockSpec((1, e), lambda i: (0, 0)),
        ],
        out_shape=[
            jax.ShapeDtypeStruct((n, 1), jnp.int32),
            jax.ShapeDtypeStruct((n, 1), jnp.int32),
            jax.ShapeDtypeStruct((n, 1), jnp.float32),
            jax.ShapeDtypeStruct((n, 1), jnp.float32),
            jax.ShapeDtypeStruct((1, e), jnp.int32),
        ],
        scratch_shapes=[pltpu.VMEM((8, e), jnp.int32)],
    )(x, w_gate)

    pcounts = ((counts[0] + tm - 1) // tm) * tm
    poffs = jnp.concatenate(
        [jnp.zeros((1,), jnp.int32), jnp.cumsum(pcounts).astype(jnp.int32)])
    tile_start = jnp.arange(tn, dtype=jnp.int32) * tm
    tile_expert = jnp.minimum(
        jnp.sum((tile_start[:, None] >= poffs[None, 1:]).astype(jnp.int32),
                axis=1), e - 1).astype(jnp.int32)
    n_active = (poffs[e] // tm).reshape(1)

    tr = 512
    pos0, pos1 = pl.pallas_call(
        _routing_body,
        grid=(n // tr,),
        in_specs=[
            pl.BlockSpec((tr, 1), lambda i: (i, 0)),
            pl.BlockSpec((tr, 1), lambda i: (i, 0)),
            pl.BlockSpec((1, e + 1), lambda i: (0, 0)),
        ],
        out_specs=[
            pl.BlockSpec((tr, 1), lambda i: (i, 0)),
            pl.BlockSpec((tr, 1), lambda i: (i, 0)),
        ],
        out_shape=[
            jax.ShapeDtypeStruct((n, 1), jnp.int32),
            jax.ShapeDtypeStruct((n, 1), jnp.int32),
        ],
        scratch_shapes=[pltpu.VMEM((8, e), jnp.int32)],
    )(i1, i2, poffs.reshape(1, e + 1))
    pos0 = pos0.reshape(n)
    pos1 = pos1.reshape(n)

    xs = _sc_scatter_rows(x, pos0, pos1, p)

    grid_spec = pltpu.PrefetchScalarGridSpec(
        num_scalar_prefetch=2,
        grid=(tn,),
        in_specs=[
            pl.BlockSpec((tm, d), lambda i, te, na: (i, 0)),
            pl.BlockSpec((1, d, h), lambda i, te, na: (te[i], 0, 0)),
            pl.BlockSpec((1, 1, h), lambda i, te, na: (te[i], 0, 0)),
            pl.BlockSpec((1, h, out_d), lambda i, te, na: (te[i], 0, 0)),
            pl.BlockSpec((1, 1, out_d), lambda i, te, na: (te[i], 0, 0)),
        ],
        out_specs=pl.BlockSpec((tm, out_d), lambda i, te, na: (i, 0)),
    )
    ys = pl.pallas_call(
        _gmm_body,
        grid_spec=grid_spec,
        out_shape=jax.ShapeDtypeStruct((p, out_d), jnp.float32),
    )(tile_expert, n_active, xs, W1, b1.reshape(e, 1, h), W2,
      b2.reshape(e, 1, out_d))

    yt = _sc_gather(ys, jnp.concatenate([pos0, pos1]))

    tb = 512
    out = pl.pallas_call(
        _combine_body,
        grid=(n // tb,),
        in_specs=[
            pl.BlockSpec((tb, out_d), lambda i: (i, 0)),
            pl.BlockSpec((tb, out_d), lambda i: (i + n // tb, 0)),
            pl.BlockSpec((tb, 1), lambda i: (i, 0)),
            pl.BlockSpec((tb, 1), lambda i: (i, 0)),
        ],
        out_specs=pl.BlockSpec((tb, out_d), lambda i: (i, 0)),
        out_shape=jax.ShapeDtypeStruct((n, out_d), jnp.float32),
    )(yt, yt, g1, g2)
    return out


# speedup vs baseline: 6.1059x; 1.0960x over previous
import functools

import jax
import jax.numpy as jnp
import numpy as np
from jax import lax
from jax.experimental import pallas as pl
from jax.experimental.pallas import tpu as pltpu
from jax.experimental.pallas import tpu_sc as plsc

_EPS = float(np.finfo(np.float64).eps)


def _gating_body(x_ref, wg_ref, i1_ref, i2_ref, g1_ref, g2_ref, cnt_ref,
                 acc_ref):
    i = pl.program_id(0)
    logits = lax.dot_general(x_ref[...], wg_ref[...], (((1,), (0,)), ((), ())),
                             preferred_element_type=jnp.float32)
    e = logits.shape[1]
    col = lax.broadcasted_iota(jnp.int32, logits.shape, 1)
    m1 = jnp.max(logits, axis=1, keepdims=True)
    i1 = jnp.min(jnp.where(logits == m1, col, e), axis=1, keepdims=True)
    masked = jnp.where(col == i1, -jnp.inf, logits)
    m2 = jnp.max(masked, axis=1, keepdims=True)
    i2 = jnp.min(jnp.where(masked == m2, col, e), axis=1, keepdims=True)
    t = jnp.exp(m2 - m1)
    denom = 1.0 + t
    i1_ref[...] = i1
    i2_ref[...] = i2
    g1_ref[...] = 1.0 / denom
    g2_ref[...] = t / denom
    hist = (jnp.sum((i1 == col[: i1.shape[0]]).astype(jnp.int32), axis=0,
                    keepdims=True)
            + jnp.sum((i2 == col[: i2.shape[0]]).astype(jnp.int32), axis=0,
                      keepdims=True))

    @pl.when(i == 0)
    def _():
        acc_ref[...] = jnp.zeros_like(acc_ref)

    acc_ref[0:1, :] += hist

    @pl.when(i == pl.num_programs(0) - 1)
    def _():
        cnt_ref[...] = acc_ref[0:1, :]


def _routing_body(i1_ref, i2_ref, poffs_ref, p0_ref, p1_ref, acc_ref):
    i = pl.program_id(0)
    tr = i1_ref.shape[0]
    ee = jnp.concatenate([i1_ref[...], i2_ref[...]], axis=0)
    e = poffs_ref.shape[1] - 1
    col = lax.broadcasted_iota(jnp.int32, (2 * tr, e), 1)
    onehot_b = ee == col
    onehot_f = onehot_b.astype(jnp.float32)

    @pl.when(i == 0)
    def _():
        acc_ref[...] = jnp.zeros_like(acc_ref)

    r = lax.broadcasted_iota(jnp.int32, (2 * tr, 2 * tr), 0)
    c = lax.broadcasted_iota(jnp.int32, (2 * tr, 2 * tr), 1)
    lt = (c < r).astype(jnp.float32)
    excl = lax.dot_general(lt, onehot_f, (((1,), (0,)), ((), ())),
                           preferred_element_type=jnp.float32)
    rank = jnp.sum(excl * onehot_f, axis=1, keepdims=True).astype(jnp.int32)
    base = jnp.sum(jnp.where(onehot_b, poffs_ref[0:1, :e], 0), axis=1,
                   keepdims=True)
    run = jnp.sum(jnp.where(onehot_b, acc_ref[0:1, :], 0), axis=1,
                  keepdims=True)
    slot = base + run + rank
    p0_ref[...] = slot[:tr]
    p1_ref[...] = slot[tr:]
    acc_ref[0:1, :] += jnp.sum(onehot_b.astype(jnp.int32), axis=0,
                               keepdims=True)


def _gmm_body(te_ref, na_ref, xs_ref, w1_ref, b1_ref, w2_ref, b2_ref, ys_ref):
    i = pl.program_id(0)

    @pl.when(i < na_ref[0])
    def _():
        h = lax.dot_general(xs_ref[...], w1_ref[0], (((1,), (0,)), ((), ())),
                            preferred_element_type=jnp.float32)
        h = jnp.maximum(h + b1_ref[0], 0.0)
        o = lax.dot_general(h, w2_ref[0], (((1,), (0,)), ((), ())),
                            preferred_element_type=jnp.float32)
        ys_ref[...] = jnp.exp(jnp.tanh(o + b2_ref[0]) * 10.0)


def _combine_body(y1_ref, y2_ref, g1_ref, g2_ref, o_ref):
    s = g1_ref[...] * y1_ref[...] + g2_ref[...] * y2_ref[...]
    o_ref[...] = jnp.log(jnp.where(s == 0.0, _EPS, s))


def _sc_scatter_rows(x, pos0, pos1, p):
    n, d = x.shape
    nw = 32
    ch = 128
    t_per_w = n // nw
    nch = t_per_w // ch
    mesh = plsc.VectorSubcoreMesh(core_axis_name="c", subcore_axis_name="s")

    @functools.partial(
        pl.kernel,
        mesh=mesh,
        out_type=jax.ShapeDtypeStruct((p, d), jnp.float32),
        scratch_types=[
            pltpu.VMEM((2, ch), jnp.int32),
            pltpu.VMEM((ch, d), jnp.float32),
            pltpu.SemaphoreType.DMA,
        ],
    )
    def scatter_k(x_hbm, p0_hbm, p1_hbm, xs_hbm, posb, rows, sem):
        wid = lax.axis_index("s") * 2 + lax.axis_index("c")
        base = wid * t_per_w

        def body(cc, carry):
            off = base + cc * ch
            pltpu.sync_copy(p0_hbm.at[pl.ds(off, ch)], posb.at[0])
            pltpu.sync_copy(p1_hbm.at[pl.ds(off, ch)], posb.at[1])
            pltpu.sync_copy(x_hbm.at[pl.ds(off, ch)], rows)
            cp0 = pltpu.async_copy(rows, xs_hbm.at[posb.at[0]], sem)
            cp1 = pltpu.async_copy(rows, xs_hbm.at[posb.at[1]], sem)
            cp0.wait()
            cp1.wait()
            return carry

        lax.fori_loop(0, nch, body, 0)

    return scatter_k(x, pos0, pos1)


def _sc_gather(table, idx):
    _, d = table.shape
    b = idx.shape[0]
    nw = 32
    ch = 128
    b_per_w = b // nw
    nch = b_per_w // ch
    mesh = plsc.VectorSubcoreMesh(core_axis_name="c", subcore_axis_name="s")

    @functools.partial(
        pl.kernel,
        mesh=mesh,
        out_type=jax.ShapeDtypeStruct((b, d), jnp.float32),
        scratch_types=[
            pltpu.VMEM((ch,), jnp.int32),
            pltpu.VMEM((ch, d), jnp.float32),
            pltpu.SemaphoreType.DMA,
        ],
    )
    def gather_k(table_hbm, idx_hbm, out_hbm, idxc, rows, sem):
        wid = lax.axis_index("s") * 2 + lax.axis_index("c")
        base = wid * b_per_w

        def body(c, carry):
            off = base + c * ch
            pltpu.sync_copy(idx_hbm.at[pl.ds(off, ch)], idxc)
            pltpu.async_copy(table_hbm.at[idxc], rows, sem).wait()
            pltpu.sync_copy(rows, out_hbm.at[pl.ds(off, ch)])
            return carry

        lax.fori_loop(0, nch, body, 0)

    return gather_k(table, idx)


def kernel(x, w_gate, W1, b1, W2, b2):
    n, d = x.shape
    e = w_gate.shape[1]
    h = W1.shape[2]
    out_d = W2.shape[2]
    tm = 256
    p = 2 * n + (tm - 1) * min(e, 2 * n)
    p = ((p + 255) // 256) * 256
    tn = p // tm

    tg = 512
    i1, i2, g1, g2, counts = pl.pallas_call(
        _gating_body,
        grid=(n // tg,),
        in_specs=[
            pl.BlockSpec((tg, d), lambda i: (i, 0)),
            pl.BlockSpec((d, e), lambda i: (0, 0)),
        ],
        out_specs=[
            pl.BlockSpec((tg, 1), lambda i: (i, 0)),
            pl.BlockSpec((tg, 1), lambda i: (i, 0)),
            pl.BlockSpec((tg, 1), lambda i: (i, 0)),
            pl.BlockSpec((tg, 1), lambda i: (i, 0)),
            pl.BlockSpec((1, e), lambda i: (0, 0)),
        ],
        out_shape=[
            jax.ShapeDtypeStruct((n, 1), jnp.int32),
            jax.ShapeDtypeStruct((n, 1), jnp.int32),
            jax.ShapeDtypeStruct((n, 1), jnp.float32),
            jax.ShapeDtypeStruct((n, 1), jnp.float32),
            jax.ShapeDtypeStruct((1, e), jnp.int32),
        ],
        scratch_shapes=[pltpu.VMEM((8, e), jnp.int32)],
    )(x, w_gate)

    pcounts = ((counts[0] + tm - 1) // tm) * tm
    poffs = jnp.concatenate(
        [jnp.zeros((1,), jnp.int32), jnp.cumsum(pcounts).astype(jnp.int32)])
    tile_start = jnp.arange(tn, dtype=jnp.int32) * tm
    tile_expert = jnp.minimum(
        jnp.sum((tile_start[:, None] >= poffs[None, 1:]).astype(jnp.int32),
                axis=1), e - 1).astype(jnp.int32)
    n_active = (poffs[e] // tm).reshape(1)

    tr = 512
    pos0, pos1 = pl.pallas_call(
        _routing_body,
        grid=(n // tr,),
        in_specs=[
            pl.BlockSpec((tr, 1), lambda i: (i, 0)),
            pl.BlockSpec((tr, 1), lambda i: (i, 0)),
            pl.BlockSpec((1, e + 1), lambda i: (0, 0)),
        ],
        out_specs=[
            pl.BlockSpec((tr, 1), lambda i: (i, 0)),
            pl.BlockSpec((tr, 1), lambda i: (i, 0)),
        ],
        out_shape=[
            jax.ShapeDtypeStruct((n, 1), jnp.int32),
            jax.ShapeDtypeStruct((n, 1), jnp.int32),
        ],
        scratch_shapes=[pltpu.VMEM((8, e), jnp.int32)],
    )(i1, i2, poffs.reshape(1, e + 1))
    pos0 = pos0.reshape(n)
    pos1 = pos1.reshape(n)

    xs = _sc_scatter_rows(x, pos0, pos1, p)

    grid_spec = pltpu.PrefetchScalarGridSpec(
        num_scalar_prefetch=2,
        grid=(tn,),
        in_specs=[
            pl.BlockSpec((tm, d), lambda i, te, na: (i, 0)),
            pl.BlockSpec((1, d, h), lambda i, te, na: (te[i], 0, 0)),
            pl.BlockSpec((1, 1, h), lambda i, te, na: (te[i], 0, 0)),
            pl.BlockSpec((1, h, out_d), lambda i, te, na: (te[i], 0, 0)),
            pl.BlockSpec((1, 1, out_d), lambda i, te, na: (te[i], 0, 0)),
        ],
        out_specs=pl.BlockSpec((tm, out_d), lambda i, te, na: (i, 0)),
    )
    ys = pl.pallas_call(
        _gmm_body,
        grid_spec=grid_spec,
        out_shape=jax.ShapeDtypeStruct((p, out_d), jnp.float32),
    )(tile_expert, n_active, xs, W1, b1.reshape(e, 1, h), W2,
      b2.reshape(e, 1, out_d))

    yt = _sc_gather(ys, jnp.concatenate([pos0, pos1]))

    tb = 512
    out = pl.pallas_call(
        _combine_body,
        grid=(n // tb,),
        in_specs=[
            pl.BlockSpec((tb, out_d), lambda i: (i, 0)),
            pl.BlockSpec((tb, out_d), lambda i: (i + n // tb, 0)),
            pl.BlockSpec((tb, 1), lambda i: (i, 0)),
            pl.BlockSpec((tb, 1), lambda i: (i, 0)),
        ],
        out_specs=pl.BlockSpec((tb, out_d), lambda i: (i, 0)),
        out_shape=jax.ShapeDtypeStruct((n, out_d), jnp.float32),
    )(yt, yt, g1, g2)
    return out
